# Initial kernel scaffold; baseline (speedup 1.0000x reference)
import jax
import jax.numpy as jnp
from jax.experimental import pallas as pl


def kernel(qry_feats, W, b, match_labels, matched_qry_ids, matched_tgt_ids, tgt_labels):
    raise NotImplementedError("write your pallas kernel here")


# speedup vs baseline: 1.6386x; 1.6386x over previous
import functools

import jax
import jax.numpy as jnp
from jax import lax
from jax.experimental import pallas as pl
from jax.experimental.pallas import tpu as pltpu
from jax.experimental.pallas import tpu_sc as plsc

NUM_LABELS = 81
ALPHA = 0.25
GAMMA = 2.0

NUM_WORKERS = 32
ROWS_PER_WORKER = 64
POS_PAD = NUM_WORKERS * ROWS_PER_WORKER
NEG_BLOCK = 2000


def _focal(logits, t):
    p = jax.nn.sigmoid(logits)
    ce = jnp.maximum(logits, 0.0) - logits * t + jnp.log1p(jnp.exp(-jnp.abs(logits)))
    p_t = p * t + (1.0 - p) * (1.0 - t)
    alpha_t = ALPHA * t + (1.0 - ALPHA) * (1.0 - t)
    om = 1.0 - p_t
    return alpha_t * ce * om * om


def _sc_gather_body(qry_hbm, ids_hbm, tidx_hbm, tlab_hbm, feats_out, tgt_out,
                    idx_v, rows_v, tidx_v, ptgt_v, sem, sem2):
    wid = lax.axis_index("s") * 2 + lax.axis_index("c")
    base = wid * ROWS_PER_WORKER
    pltpu.sync_copy(ids_hbm.at[pl.ds(base, ROWS_PER_WORKER)], idx_v)
    pltpu.sync_copy(tidx_hbm.at[pl.ds(base, ROWS_PER_WORKER)], tidx_v)
    cp1 = pltpu.async_copy(qry_hbm.at[idx_v], rows_v, sem)
    cp2 = pltpu.async_copy(tlab_hbm.at[tidx_v], ptgt_v, sem2)
    cp1.wait()
    cp2.wait()
    pltpu.sync_copy(rows_v, feats_out.at[pl.ds(base, ROWS_PER_WORKER)])
    pltpu.sync_copy(ptgt_v, tgt_out.at[pl.ds(base, ROWS_PER_WORKER)])


def _sc_gather(qry_feats, ids_pad, tidx_pad, tgt_labels):
    d = qry_feats.shape[1]
    num_tgts = tgt_labels.shape[0]
    mesh = plsc.VectorSubcoreMesh(core_axis_name="c", subcore_axis_name="s")
    return pl.kernel(
        _sc_gather_body,
        out_type=[
            jax.ShapeDtypeStruct((POS_PAD, d), jnp.float32),
            jax.ShapeDtypeStruct((POS_PAD,), jnp.int32),
        ],
        mesh=mesh,
        scratch_types=[
            pltpu.VMEM((ROWS_PER_WORKER,), jnp.int32),
            pltpu.VMEM((ROWS_PER_WORKER, d), jnp.float32),
            pltpu.VMEM((ROWS_PER_WORKER,), jnp.int32),
            pltpu.VMEM((ROWS_PER_WORKER,), jnp.int32),
            pltpu.SemaphoreType.DMA,
            pltpu.SemaphoreType.DMA,
        ],
    )(qry_feats, ids_pad, tidx_pad, tgt_labels)


def _neg_body(x_ref, w_ref, b_ref, ml_ref, out_ref):
    @pl.when(pl.program_id(0) == 0)
    def _init():
        out_ref[0, 0] = 0.0

    logits = jnp.dot(x_ref[...], w_ref[...], preferred_element_type=jnp.float32)
    logits = logits + b_ref[...]
    cols = lax.broadcasted_iota(jnp.int32, (1, NUM_LABELS), 1)
    t = (cols == NUM_LABELS - 1).astype(jnp.float32)
    loss = _focal(logits, t)
    w = (ml_ref[...] == 0).astype(jnp.float32)
    out_ref[0, 0] += jnp.sum(loss * w)


def _pos_body(x_ref, w_ref, b_ref, tgt_ref, pw_ref, out_ref):
    logits = jnp.dot(x_ref[...], w_ref[...], preferred_element_type=jnp.float32)
    logits = logits + b_ref[...]
    cols = lax.broadcasted_iota(jnp.int32, (1, NUM_LABELS), 1)
    t = (cols == tgt_ref[...]).astype(jnp.float32)
    loss = _focal(logits, t)
    out_ref[0, 0] = jnp.sum(loss * pw_ref[...])


def kernel(qry_feats, W, b, match_labels, matched_qry_ids, matched_tgt_ids, tgt_labels):
    num_qrys, d = qry_feats.shape
    num_pos = matched_qry_ids.shape[0]
    pad = POS_PAD - num_pos

    ids_pad = jnp.concatenate(
        [matched_qry_ids.astype(jnp.int32), jnp.zeros((pad,), jnp.int32)])
    tidx_pad = jnp.concatenate(
        [matched_tgt_ids.astype(jnp.int32), jnp.zeros((pad,), jnp.int32)])

    pos_feats, pos_tgt = _sc_gather(
        qry_feats, ids_pad, tidx_pad, tgt_labels.astype(jnp.int32))

    b2 = b.reshape(1, NUM_LABELS)
    ml2 = match_labels.astype(jnp.int32).reshape(num_qrys, 1)
    grid = num_qrys // NEG_BLOCK

    neg_sum = pl.pallas_call(
        _neg_body,
        grid=(grid,),
        in_specs=[
            pl.BlockSpec((NEG_BLOCK, d), lambda i: (i, 0)),
            pl.BlockSpec((d, NUM_LABELS), lambda i: (0, 0)),
            pl.BlockSpec((1, NUM_LABELS), lambda i: (0, 0)),
            pl.BlockSpec((NEG_BLOCK, 1), lambda i: (i, 0)),
        ],
        out_specs=pl.BlockSpec((1, 1), lambda i: (0, 0), memory_space=pltpu.SMEM),
        out_shape=jax.ShapeDtypeStruct((1, 1), jnp.float32),
    )(qry_feats, W, b2, ml2)

    pos_w = jnp.concatenate(
        [jnp.ones((num_pos,), jnp.float32), jnp.zeros((pad,), jnp.float32)]
    ).reshape(POS_PAD, 1)
    tgt2 = pos_tgt.reshape(POS_PAD, 1)

    pos_sum = pl.pallas_call(
        _pos_body,
        in_specs=[
            pl.BlockSpec((POS_PAD, d), lambda: (0, 0)),
            pl.BlockSpec((d, NUM_LABELS), lambda: (0, 0)),
            pl.BlockSpec((1, NUM_LABELS), lambda: (0, 0)),
            pl.BlockSpec((POS_PAD, 1), lambda: (0, 0)),
            pl.BlockSpec((POS_PAD, 1), lambda: (0, 0)),
        ],
        out_specs=pl.BlockSpec((1, 1), lambda: (0, 0), memory_space=pltpu.SMEM),
        out_shape=jax.ShapeDtypeStruct((1, 1), jnp.float32),
    )(pos_feats, W, b2, tgt2, pos_w)

    avg_factor = jnp.float32(max(num_pos, 1))
    return (neg_sum[0, 0] + pos_sum[0, 0]) / avg_factor


# speedup vs baseline: 2.6412x; 1.6119x over previous
import jax
import jax.numpy as jnp
from jax import lax
from jax.experimental import pallas as pl
from jax.experimental.pallas import tpu as pltpu
from jax.experimental.pallas import tpu_sc as plsc

NUM_LABELS = 81
ALPHA = 0.25
GAMMA = 2.0

SC_WORKERS = 25
ROWS_PER_WORKER = 80
NEG_BLOCK = 3000


def _focal_t(logits_t, t):
    a = jnp.abs(logits_t)
    e = jnp.exp(-a)
    u = 1.0 + e
    r = 1.0 / u
    s = jnp.maximum(logits_t, 0.0) + jnp.log(u)
    nonneg = logits_t >= 0.0
    p = jnp.where(nonneg, r, 1.0 - r)
    ce = jnp.where(t > 0.0, s - logits_t, s)
    p_t = jnp.where(t > 0.0, p, 1.0 - p)
    alpha_t = jnp.where(t > 0.0, ALPHA, 1.0 - ALPHA)
    om = 1.0 - p_t
    return alpha_t * ce * om * om


def _sc_gather_body(qry_hbm, ids_hbm, tidx_hbm, tlab_hbm, feats_out, tgt_out,
                    idx_v, rows_v, tidx_v, ptgt_v, sem, sem2):
    wid = lax.axis_index("s") * 2 + lax.axis_index("c")

    @pl.when(wid < SC_WORKERS)
    def _():
        base = wid * ROWS_PER_WORKER
        pltpu.sync_copy(ids_hbm.at[pl.ds(base, ROWS_PER_WORKER)], idx_v)
        pltpu.sync_copy(tidx_hbm.at[pl.ds(base, ROWS_PER_WORKER)], tidx_v)
        cp1 = pltpu.async_copy(qry_hbm.at[idx_v], rows_v, sem)
        cp2 = pltpu.async_copy(tlab_hbm.at[tidx_v], ptgt_v, sem2)
        cp1.wait()
        cp2.wait()
        pltpu.sync_copy(rows_v, feats_out.at[pl.ds(base, ROWS_PER_WORKER)])
        pltpu.sync_copy(ptgt_v, tgt_out.at[pl.ds(base, ROWS_PER_WORKER)])


def _sc_gather(qry_feats, ids, tidx, tgt_labels):
    d = qry_feats.shape[1]
    num_pos = ids.shape[0]
    mesh = plsc.VectorSubcoreMesh(core_axis_name="c", subcore_axis_name="s")
    return pl.kernel(
        _sc_gather_body,
        out_type=[
            jax.ShapeDtypeStruct((num_pos, d), jnp.float32),
            jax.ShapeDtypeStruct((num_pos,), jnp.int32),
        ],
        mesh=mesh,
        scratch_types=[
            pltpu.VMEM((ROWS_PER_WORKER,), jnp.int32),
            pltpu.VMEM((ROWS_PER_WORKER, d), jnp.float32),
            pltpu.VMEM((ROWS_PER_WORKER,), jnp.int32),
            pltpu.VMEM((ROWS_PER_WORKER,), jnp.int32),
            pltpu.SemaphoreType.DMA,
            pltpu.SemaphoreType.DMA,
        ],
    )(qry_feats, ids, tidx, tgt_labels)


def _neg_body(x_ref, w_ref, b_ref, ml_ref, out_ref):
    @pl.when(pl.program_id(0) == 0)
    def _init():
        out_ref[0, 0] = 0.0

    logits_t = lax.dot_general(
        w_ref[...], x_ref[...], (((0,), (1,)), ((), ())),
        preferred_element_type=jnp.float32)
    logits_t = logits_t + b_ref[...]
    rows = lax.broadcasted_iota(jnp.int32, (NUM_LABELS, 1), 0)
    t = (rows == NUM_LABELS - 1).astype(jnp.float32)
    loss = _focal_t(logits_t, t)
    w = (ml_ref[0] == 0).astype(jnp.float32)
    out_ref[0, 0] += jnp.sum(jnp.sum(loss, axis=0, keepdims=True) * w)


def _pos_body(x_ref, w_ref, b_ref, tgt_ref, out_ref):
    logits_t = lax.dot_general(
        w_ref[...], x_ref[...], (((0,), (1,)), ((), ())),
        preferred_element_type=jnp.float32)
    logits_t = logits_t + b_ref[...]
    rows = lax.broadcasted_iota(jnp.int32, (NUM_LABELS, 1), 0)
    t = (rows == tgt_ref[...]).astype(jnp.float32)
    loss = _focal_t(logits_t, t)
    out_ref[0, 0] = jnp.sum(loss)


def kernel(qry_feats, W, b, match_labels, matched_qry_ids, matched_tgt_ids, tgt_labels):
    num_qrys, d = qry_feats.shape
    num_pos = matched_qry_ids.shape[0]

    pos_feats, pos_tgt = _sc_gather(
        qry_feats, matched_qry_ids.astype(jnp.int32),
        matched_tgt_ids.astype(jnp.int32), tgt_labels.astype(jnp.int32))

    b2 = b.reshape(NUM_LABELS, 1)
    grid = num_qrys // NEG_BLOCK
    ml3 = match_labels.astype(jnp.int32).reshape(grid, 1, NEG_BLOCK)

    neg_sum = pl.pallas_call(
        _neg_body,
        grid=(grid,),
        in_specs=[
            pl.BlockSpec((NEG_BLOCK, d), lambda i: (i, 0)),
            pl.BlockSpec((d, NUM_LABELS), lambda i: (0, 0)),
            pl.BlockSpec((NUM_LABELS, 1), lambda i: (0, 0)),
            pl.BlockSpec((1, 1, NEG_BLOCK), lambda i: (i, 0, 0)),
        ],
        out_specs=pl.BlockSpec((1, 1), lambda i: (0, 0), memory_space=pltpu.SMEM),
        out_shape=jax.ShapeDtypeStruct((1, 1), jnp.float32),
    )(qry_feats, W, b2, ml3)

    tgt2 = pos_tgt.reshape(1, num_pos)

    pos_sum = pl.pallas_call(
        _pos_body,
        in_specs=[
            pl.BlockSpec((num_pos, d), lambda: (0, 0)),
            pl.BlockSpec((d, NUM_LABELS), lambda: (0, 0)),
            pl.BlockSpec((NUM_LABELS, 1), lambda: (0, 0)),
            pl.BlockSpec((1, num_pos), lambda: (0, 0)),
        ],
        out_specs=pl.BlockSpec((1, 1), lambda: (0, 0), memory_space=pltpu.SMEM),
        out_shape=jax.ShapeDtypeStruct((1, 1), jnp.float32),
    )(pos_feats, W, b2, tgt2)

    avg_factor = jnp.float32(max(num_pos, 1))
    return (neg_sum[0, 0] + pos_sum[0, 0]) / avg_factor
